# baseline (speedup 1.0000x reference)
import functools

import jax
import jax.numpy as jnp
from jax import lax
from jax.experimental import pallas as pl
from jax.experimental.pallas import tpu as pltpu
from jax.experimental.pallas import tpu_sc as plsc

D = 64
NC = 2
NS = 16
NW = NC * NS
CHUNK = 128


def _sc_gather(idx3, table):
    n_chunks = idx3.shape[1]
    B = NW * n_chunks * CHUNK
    mesh = plsc.VectorSubcoreMesh(core_axis_name="c", subcore_axis_name="s")

    @functools.partial(
        pl.kernel,
        out_type=jax.ShapeDtypeStruct((B, D), jnp.float32),
        mesh=mesh,
        scratch_types=[
            pltpu.VMEM((n_chunks, CHUNK), jnp.int32),
            pltpu.VMEM((CHUNK, D), jnp.float32),
            pltpu.SemaphoreType.DMA,
        ],
        compiler_params=pltpu.CompilerParams(use_tc_tiling_on_sc=False),
    )
    def k(idx_hbm, table_hbm, out_hbm, idx_v, rows_v, sem):
        wid = lax.axis_index("s") * NC + lax.axis_index("c")
        base = wid * (n_chunks * CHUNK)
        pltpu.sync_copy(idx_hbm.at[wid], idx_v)

        def body(j, _):
            pltpu.async_copy(table_hbm.at[idx_v.at[j]], rows_v, sem).wait()
            pltpu.sync_copy(rows_v, out_hbm.at[pl.ds(base + j * CHUNK, CHUNK)])
            return 0

        lax.fori_loop(0, n_chunks, body, 0)

    return k(idx3, table)


def kernel(inp, concepts):
    orig_shape = inp.shape
    flat = inp.reshape(-1).astype(jnp.int32)
    B = flat.shape[0]
    assert B % (NW * CHUNK) == 0, B
    idx3 = flat.reshape(NW, B // (NW * CHUNK), CHUNK)
    out = _sc_gather(idx3, concepts)
    return out.reshape(*orig_shape, D)


# speedup vs baseline: 1.1137x; 1.1137x over previous
import functools

import jax
import jax.numpy as jnp
from jax import lax
from jax.experimental import pallas as pl
from jax.experimental.pallas import tpu as pltpu
from jax.experimental.pallas import tpu_sc as plsc

D = 64
NC = 2
NS = 16
NW = NC * NS
CHUNK = 128
G = 4


def _sc_gather(idx3, table):
    n_chunks = idx3.shape[1]
    assert n_chunks % (2 * G) == 0, n_chunks
    ngroups = n_chunks // G
    per_w = n_chunks * CHUNK
    B = NW * per_w
    mesh = plsc.VectorSubcoreMesh(core_axis_name="c", subcore_axis_name="s")

    @functools.partial(
        pl.kernel,
        out_type=jax.ShapeDtypeStruct((B, D), jnp.float32),
        mesh=mesh,
        scratch_types=[
            pltpu.VMEM((n_chunks, CHUNK), jnp.int32),
            pltpu.VMEM((2 * G, CHUNK, D), jnp.float32),
            pltpu.SemaphoreType.DMA,
            pltpu.SemaphoreType.DMA,
        ],
        compiler_params=pltpu.CompilerParams(use_tc_tiling_on_sc=False),
    )
    def k(idx_hbm, table_hbm, out_hbm, idx_v, rows_v, gsem, osem):
        wid = lax.axis_index("s") * NC + lax.axis_index("c")
        base = wid * per_w
        pltpu.sync_copy(idx_hbm.at[wid], idx_v)

        def start_gather(j, slot):
            pltpu.make_async_copy(
                table_hbm.at[idx_v.at[j]], rows_v.at[slot], gsem).start()

        def wait_gather(slot):
            pltpu.make_async_copy(
                table_hbm.at[pl.ds(0, CHUNK)], rows_v.at[slot], gsem).wait()

        def start_out(j, slot):
            pltpu.make_async_copy(
                rows_v.at[slot],
                out_hbm.at[pl.ds(base + j * CHUNK, CHUNK)], osem).start()

        def wait_out(slot):
            pltpu.make_async_copy(
                rows_v.at[slot], out_hbm.at[pl.ds(base, CHUNK)], osem).wait()

        for b in range(G):
            start_gather(b, b)

        def body(t, _):
            for parity in (0, 1):
                g = 2 * t + parity
                s0 = parity * G
                op = (1 - parity) * G
                for b in range(G):
                    wait_gather(s0 + b)
                for b in range(G):
                    start_out(g * G + b, s0 + b)

                @pl.when(g > 0)
                def _():
                    for b in range(G):
                        wait_out(op + b)

                @pl.when(g + 1 < ngroups)
                def _():
                    for b in range(G):
                        start_gather((g + 1) * G + b, op + b)
            return 0

        lax.fori_loop(0, ngroups // 2, body, 0)
        for b in range(G):
            wait_out(G + b)

    return k(idx3, table)


def kernel(inp, concepts):
    orig_shape = inp.shape
    flat = inp.reshape(-1).astype(jnp.int32)
    B = flat.shape[0]
    assert B % (NW * CHUNK) == 0, B
    idx3 = flat.reshape(NW, B // (NW * CHUNK), CHUNK)
    out = _sc_gather(idx3, concepts)
    return out.reshape(*orig_shape, D)


# speedup vs baseline: 1.1147x; 1.0009x over previous
import functools

import jax
import jax.numpy as jnp
from jax import lax
from jax.experimental import pallas as pl
from jax.experimental.pallas import tpu as pltpu
from jax.experimental.pallas import tpu_sc as plsc

D = 64
NC = 2
NS = 16
NW = NC * NS
CHUNK = 512
G = 1


def _sc_gather(idx3, table):
    n_chunks = idx3.shape[1]
    assert n_chunks % (2 * G) == 0, n_chunks
    ngroups = n_chunks // G
    per_w = n_chunks * CHUNK
    B = NW * per_w
    mesh = plsc.VectorSubcoreMesh(core_axis_name="c", subcore_axis_name="s")

    @functools.partial(
        pl.kernel,
        out_type=jax.ShapeDtypeStruct((B, D), jnp.float32),
        mesh=mesh,
        scratch_types=[
            pltpu.VMEM((n_chunks, CHUNK), jnp.int32),
            pltpu.VMEM((2 * G, CHUNK, D), jnp.float32),
            pltpu.SemaphoreType.DMA,
            pltpu.SemaphoreType.DMA,
        ],
        compiler_params=pltpu.CompilerParams(use_tc_tiling_on_sc=False),
    )
    def k(idx_hbm, table_hbm, out_hbm, idx_v, rows_v, gsem, osem):
        wid = lax.axis_index("s") * NC + lax.axis_index("c")
        base = wid * per_w
        pltpu.sync_copy(idx_hbm.at[wid], idx_v)

        def start_gather(j, slot):
            pltpu.make_async_copy(
                table_hbm.at[idx_v.at[j]], rows_v.at[slot], gsem).start()

        def wait_gather(slot):
            pltpu.make_async_copy(
                table_hbm.at[pl.ds(0, CHUNK)], rows_v.at[slot], gsem).wait()

        def start_out(j, slot):
            pltpu.make_async_copy(
                rows_v.at[slot],
                out_hbm.at[pl.ds(base + j * CHUNK, CHUNK)], osem).start()

        def wait_out(slot):
            pltpu.make_async_copy(
                rows_v.at[slot], out_hbm.at[pl.ds(base, CHUNK)], osem).wait()

        for b in range(G):
            start_gather(b, b)

        def body(t, _):
            for parity in (0, 1):
                g = 2 * t + parity
                s0 = parity * G
                op = (1 - parity) * G
                for b in range(G):
                    wait_gather(s0 + b)
                for b in range(G):
                    start_out(g * G + b, s0 + b)

                @pl.when(g > 0)
                def _():
                    for b in range(G):
                        wait_out(op + b)

                @pl.when(g + 1 < ngroups)
                def _():
                    for b in range(G):
                        start_gather((g + 1) * G + b, op + b)
            return 0

        lax.fori_loop(0, ngroups // 2, body, 0)
        for b in range(G):
            wait_out(G + b)

    return k(idx3, table)


def kernel(inp, concepts):
    orig_shape = inp.shape
    flat = inp.reshape(-1).astype(jnp.int32)
    B = flat.shape[0]
    assert B % (NW * CHUNK) == 0, B
    idx3 = flat.reshape(NW, B // (NW * CHUNK), CHUNK)
    out = _sc_gather(idx3, concepts)
    return out.reshape(*orig_shape, D)


# speedup vs baseline: 1.4829x; 1.3303x over previous
import functools

import jax
import jax.numpy as jnp
from jax import lax
from jax.experimental import pallas as pl
from jax.experimental.pallas import tpu as pltpu
from jax.experimental.pallas import tpu_sc as plsc

D = 64
NC = 2
NS = 16
NW = NC * NS
CHUNK = 512
G = 1


def _sc_gather(idx3, table):
    n_chunks = idx3.shape[1]
    assert n_chunks % (2 * G) == 0, n_chunks
    ngroups = n_chunks // G
    per_w = n_chunks * CHUNK
    B = NW * per_w
    mesh = plsc.VectorSubcoreMesh(core_axis_name="c", subcore_axis_name="s")

    @functools.partial(
        pl.kernel,
        out_type=jax.ShapeDtypeStruct((B, 128), jnp.float32),
        mesh=mesh,
        scratch_types=[
            pltpu.VMEM((n_chunks, CHUNK), jnp.int32),
            pltpu.VMEM((2 * G, CHUNK, D), jnp.float32),
            pltpu.SemaphoreType.DMA,
            pltpu.SemaphoreType.DMA,
        ],
        compiler_params=pltpu.CompilerParams(use_tc_tiling_on_sc=False),
    )
    def k(idx_hbm, table_hbm, out_hbm, idx_v, rows_v, gsem, osem):
        wid = lax.axis_index("s") * NC + lax.axis_index("c")
        base = wid * per_w
        pltpu.sync_copy(idx_hbm.at[wid], idx_v)

        def start_gather(j, slot):
            pltpu.make_async_copy(
                table_hbm.at[idx_v.at[j]], rows_v.at[slot], gsem).start()

        def wait_gather(slot):
            pltpu.make_async_copy(
                table_hbm.at[pl.ds(0, CHUNK)], rows_v.at[slot], gsem).wait()

        def start_out(j, slot):
            pltpu.make_async_copy(
                rows_v.at[slot],
                out_hbm.at[pl.ds(base + j * CHUNK, CHUNK), pl.ds(0, D)],
                osem).start()

        def wait_out(slot):
            pltpu.make_async_copy(
                rows_v.at[slot],
                out_hbm.at[pl.ds(base, CHUNK), pl.ds(0, D)], osem).wait()

        for b in range(G):
            start_gather(b, b)

        def body(t, _):
            for parity in (0, 1):
                g = 2 * t + parity
                s0 = parity * G
                op = (1 - parity) * G
                for b in range(G):
                    wait_gather(s0 + b)
                for b in range(G):
                    start_out(g * G + b, s0 + b)

                @pl.when(g > 0)
                def _():
                    for b in range(G):
                        wait_out(op + b)

                @pl.when(g + 1 < ngroups)
                def _():
                    for b in range(G):
                        start_gather((g + 1) * G + b, op + b)
            return 0

        lax.fori_loop(0, ngroups // 2, body, 0)
        for b in range(G):
            wait_out(G + b)

    return k(idx3, table)


def kernel(inp, concepts):
    orig_shape = inp.shape
    flat = inp.reshape(-1).astype(jnp.int32)
    B = flat.shape[0]
    assert B % (NW * CHUNK) == 0, B
    idx3 = flat.reshape(NW, B // (NW * CHUNK), CHUNK)
    out = _sc_gather(idx3, concepts)
    return out.reshape(*orig_shape, 128)[..., :D]
